# baseline (speedup 1.0000x reference)
import functools

import jax
import jax.numpy as jnp
from jax.experimental import pallas as pl
from jax.experimental.pallas import tpu as pltpu

_N = 2097152
_ROWS = 8192
_ELEM_COLS = 1024
_LBL_COLS = 256
_BLK = 512
_GRID = _ROWS // _BLK


def _loss_kernel(o_ref, t_ref, l_ref, out_ref, acc_ref):
    step = pl.program_id(0)

    @pl.when(step == 0)
    def _init():
        acc_ref[0] = 0.0
        acc_ref[1] = 0.0

    diff = o_ref[...] - t_ref[...]
    absd = jnp.abs(diff)
    per_elem = jnp.where(absd < 1.0, 0.5 * diff * diff, absd - 0.5)

    ji = jax.lax.broadcasted_iota(jnp.int32, (_ELEM_COLS, _LBL_COLS), 0)
    ii = jax.lax.broadcasted_iota(jnp.int32, (_ELEM_COLS, _LBL_COLS), 1)
    sel = ((ji // 4) == ii).astype(jnp.float32)
    rowsum = jax.lax.dot(per_elem, sel, preferred_element_type=jnp.float32)

    m = (l_ref[...] == 1).astype(jnp.float32)
    acc_ref[0] += jnp.sum(rowsum * m)
    acc_ref[1] += jnp.sum(m)

    @pl.when(step == _GRID - 1)
    def _fini():
        denom = jnp.maximum(acc_ref[1] * 4.0, 1.0)
        out_ref[0] = acc_ref[0] / denom


@jax.jit
def kernel(out_bbox, labels, bbox_targets):
    o = out_bbox.reshape(_ROWS, _ELEM_COLS)
    t = bbox_targets.reshape(_ROWS, _ELEM_COLS)
    l = labels.reshape(_ROWS, _LBL_COLS)

    out = pl.pallas_call(
        _loss_kernel,
        grid=(_GRID,),
        in_specs=[
            pl.BlockSpec((_BLK, _ELEM_COLS), lambda i: (i, 0)),
            pl.BlockSpec((_BLK, _ELEM_COLS), lambda i: (i, 0)),
            pl.BlockSpec((_BLK, _LBL_COLS), lambda i: (i, 0)),
        ],
        out_specs=pl.BlockSpec(memory_space=pltpu.SMEM),
        out_shape=jax.ShapeDtypeStruct((1,), jnp.float32),
        scratch_shapes=[pltpu.SMEM((2,), jnp.float32)],
    )(o, t, l)
    return out[0]


# speedup vs baseline: 3.2284x; 3.2284x over previous
import jax
import jax.numpy as jnp
from jax.experimental import pallas as pl
from jax.experimental.pallas import tpu as pltpu

_N = 2097152
_BLK = 16384
_GRID = _N // _BLK


def _loss_kernel(o_ref, t_ref, l_ref, out_ref, acc_ref):
    step = pl.program_id(0)

    @pl.when(step == 0)
    def _init():
        acc_ref[0] = 0.0
        acc_ref[1] = 0.0

    diff = o_ref[...] - t_ref[...]
    absd = jnp.abs(diff)
    per_elem = jnp.where(absd < 1.0, 0.5 * diff * diff, absd - 0.5)

    maskf = (l_ref[...] == 1).astype(jnp.float32)
    acc_ref[0] += jnp.sum(per_elem * maskf[:, None])
    acc_ref[1] += jnp.sum(maskf)

    @pl.when(step == _GRID - 1)
    def _fini():
        denom = jnp.maximum(acc_ref[1] * 4.0, 1.0)
        out_ref[0] = acc_ref[0] / denom


@jax.jit
def kernel(out_bbox, labels, bbox_targets):
    out = pl.pallas_call(
        _loss_kernel,
        grid=(_GRID,),
        in_specs=[
            pl.BlockSpec((_BLK, 4), lambda i: (i, 0)),
            pl.BlockSpec((_BLK, 4), lambda i: (i, 0)),
            pl.BlockSpec((_BLK,), lambda i: (i,)),
        ],
        out_specs=pl.BlockSpec(memory_space=pltpu.SMEM),
        out_shape=jax.ShapeDtypeStruct((1,), jnp.float32),
        scratch_shapes=[pltpu.SMEM((2,), jnp.float32)],
    )(out_bbox, bbox_targets, labels)
    return out[0]
